# Initial kernel scaffold; baseline (speedup 1.0000x reference)
import jax
import jax.numpy as jnp
from jax.experimental import pallas as pl


def kernel(x, qp_w1, qp_b1, qp_w2, qp_b2, kp_w1, kp_b1, kp_w2, kp_b2, vp_w1, vp_b1, vp_w2, vp_b2, out_w, out_b, ln1_g, ln1_b, ln2_g, ln2_b, gate_w, gate_b, ew1, eb1, ew2, eb2, ew3, eb3):
    raise NotImplementedError("write your pallas kernel here")


# speedup vs baseline: 1.3426x; 1.3426x over previous
import functools

import jax
import jax.numpy as jnp
from jax import lax
from jax.experimental import pallas as pl
from jax.experimental.pallas import tpu as pltpu
from jax.experimental.pallas import tpu_sc as plsc

S, D, H, DK, DFF, E = 2048, 768, 12, 64, 3072, 8
T = 128
NPAD = S + E * T
NBLK = NPAD // T
CH = 512
NCH = DFF // CH
F32 = jnp.float32
BF16 = jnp.bfloat16

_PC = pl.pallas_call


def _fiota(shape, dim):
    return lax.broadcasted_iota(jnp.int32, shape, dim).astype(F32)

def _ln(x, g, b):
    m = jnp.mean(x, axis=-1, keepdims=True)
    v = jnp.mean((x - m) ** 2, axis=-1, keepdims=True)
    return (x - m) * lax.rsqrt(v + 1e-5) * g + b


def _dot_nt(a, b):
    return lax.dot_general(a, b, (((1,), (1,)), ((), ())),
                           preferred_element_type=F32)


def _dot_nn(a, b):
    return lax.dot_general(a, b, (((1,), (0,)), ((), ())),
                           preferred_element_type=F32)


def _silu(x):
    return x * jax.nn.sigmoid(x)


def _qkv_body(x_ref, ln1g_ref, ln1b_ref,
              qw1_ref, qb1_ref, qw2_ref, qb2_ref,
              kw1_ref, kb1_ref, kw2_ref, kb2_ref,
              vw1_ref, vb1_ref, vw2_ref, vb2_ref,
              pq_ref, pk_ref, v_ref):
    x2 = _ln(x_ref[...], ln1g_ref[...], ln1b_ref[...])
    q = _silu(_dot_nt(x2, qw1_ref[...]) + qb1_ref[...]) * (
        _dot_nt(x2, qw2_ref[...]) + qb2_ref[...])
    k = _silu(_dot_nt(x2, kw1_ref[...]) + kb1_ref[...]) * (
        _dot_nt(x2, kw2_ref[...]) + kb2_ref[...])
    v = _silu(_dot_nt(x2, vw1_ref[...]) + vb1_ref[...]) * (
        _dot_nt(x2, vw2_ref[...]) + vb2_ref[...])
    pq_ref[...] = jnp.where(q > 0, q + 1.0, jnp.exp(q))
    pk_ref[...] = jnp.where(k > 0, k + 1.0, jnp.exp(k))
    v_ref[...] = v


def _qkv(x, ln1g, ln1b, qw1, qb1, qw2, qb2, kw1, kb1, kw2, kb2,
         vw1, vb1, vw2, vb2):
    tb = 256
    row = pl.BlockSpec((tb, D), lambda i: (i, 0))
    wfull = pl.BlockSpec((D, D), lambda i: (0, 0))
    bfull = pl.BlockSpec((1, D), lambda i: (0, 0))
    return _PC(
        _qkv_body,
        grid=(S // tb,),
        in_specs=[row, bfull, bfull,
                  wfull, bfull, wfull, bfull,
                  wfull, bfull, wfull, bfull,
                  wfull, bfull, wfull, bfull],
        out_specs=[row, row, row],
        out_shape=[jax.ShapeDtypeStruct((S, D), F32)] * 3,
    )(x, ln1g, ln1b, qw1, qb1, qw2, qb2, kw1, kb1, kw2, kb2,
      vw1, vb1, vw2, vb2)


def _attn_body(pq_ref, pk_ref, v_ref, o_ref):
    for h in range(H):
        sl = slice(h * DK, (h + 1) * DK)
        pq = pq_ref[:, sl]
        pk = pk_ref[:, sl]
        v = v_ref[:, sl]
        kv = lax.dot_general(pk, v, (((0,), (0,)), ((), ())),
                             preferred_element_type=F32)
        ksum = jnp.sum(pk, axis=0, keepdims=True)
        qks = jnp.sum(pq * ksum, axis=1, keepdims=True)
        o_ref[:, sl] = _dot_nn(pq, kv) / (qks + 1e-6)


def _attn(pq, pk, v):
    full = pl.BlockSpec((S, D), lambda: (0, 0))
    return _PC(
        _attn_body,
        in_specs=[full, full, full],
        out_specs=full,
        out_shape=jax.ShapeDtypeStruct((S, D), F32),
    )(pq, pk, v)


def _route_body(x_ref, attn_ref, ow_ref, ob_ref, ln2g_ref, ln2b_ref,
                gw_ref, gb_ref,
                y_ref, x2b_ref, slot_ref, prob_ref, tos_ref, be_ref,
                ranks_ref):
    y = x_ref[...] + _dot_nt(attn_ref[...], ow_ref[...]) + ob_ref[...]
    y_ref[...] = y
    x2b = _ln(y, ln2g_ref[...], ln2b_ref[...])
    x2b_ref[...] = x2b
    logits = _dot_nt(x2b, gw_ref[...]) + gb_ref[...]
    mx = jnp.max(logits, axis=1, keepdims=True)
    iota_e = _fiota((S, E), 1)
    e_idx = jnp.min(jnp.where(logits >= mx, iota_e, float(E)),
                    axis=1, keepdims=True)
    ssum = jnp.sum(jnp.exp(logits - mx), axis=1, keepdims=True)
    topv = 1.0 / ssum
    prob_ref[...] = topv / (topv + 1e-6)
    onehot = (iota_e == e_idx).astype(F32)

    rchunk = 256
    def rank_step(ci, _):
        r0 = ci * rchunk
        row_i = r0 + _fiota((rchunk, S), 0)
        col_j = _fiota((rchunk, S), 1)
        tri = (col_j < row_i).astype(F32)
        ranks_ref[pl.ds(r0, rchunk), :] = _dot_nn(tri, onehot)
        return 0
    lax.fori_loop(0, S // rchunk, rank_step, 0)
    ranks8 = ranks_ref[...]
    rank = jnp.sum(ranks8 * onehot, axis=1, keepdims=True)

    counts = jnp.sum(onehot, axis=0, keepdims=True)
    pc = jnp.floor((counts + (T - 1.0)) * (1.0 / T)) * T
    lt = (_fiota((E, E), 0) <
          _fiota((E, E), 1)).astype(F32)
    off = _dot_nn(pc, lt)
    off_tok = jnp.sum(onehot * off, axis=1, keepdims=True)
    slot = off_tok + rank
    slot_ref[...] = slot.astype(jnp.int32)

    def tos_step(b, _):
        sl_ids = b * T + _fiota((1, T), 1)
        m = slot == sl_ids
        tok_i = _fiota((S, T), 0)
        tos = jnp.sum(jnp.where(m, tok_i, 0.0), axis=0, keepdims=True)
        tos_ref[pl.ds(b, 1), :] = tos.astype(jnp.int32)
        return 0
    lax.fori_loop(0, NBLK, tos_step, 0)

    bstart = _fiota((NBLK, E), 0) * T
    cond = (bstart >= off) & (bstart < off + pc)
    be = jnp.sum(jnp.where(cond, _fiota((NBLK, E), 1), 0.0),
                 axis=1, keepdims=True)
    be_ref[...] = be.astype(jnp.int32)


def _route(x, attn, ow, ob, ln2g, ln2b, gw, gb):
    full = lambda shape: pl.BlockSpec(shape, lambda: tuple(0 for _ in shape))
    return _PC(
        _route_body,
        in_specs=[full((S, D)), full((S, D)), full((D, D)), full((1, D)),
                  full((1, D)), full((1, D)), full((E, D)), full((1, E))],
        out_specs=[full((S, D)), full((S, D)), full((S, 1)), full((S, 1)),
                   full((NBLK, T)), full((NBLK, 1))],
        out_shape=[jax.ShapeDtypeStruct((S, D), F32),
                   jax.ShapeDtypeStruct((S, D), F32),
                   jax.ShapeDtypeStruct((S, 1), jnp.int32),
                   jax.ShapeDtypeStruct((S, 1), F32),
                   jax.ShapeDtypeStruct((NBLK, T), jnp.int32),
                   jax.ShapeDtypeStruct((NBLK, 1), jnp.int32)],
        scratch_shapes=[pltpu.VMEM((S, E), F32)],
    )(x, attn, ow, ob, ln2g, ln2b, gw, gb)


def _sc_row_gather(table, idx):
    _, d = table.shape
    b = idx.shape[0]
    nw = 32
    bpw = b // nw
    mesh = plsc.VectorSubcoreMesh(core_axis_name="c", subcore_axis_name="s")

    @functools.partial(
        pl.kernel,
        out_type=jax.ShapeDtypeStruct((b, d), table.dtype),
        mesh=mesh,
        scratch_types=[pltpu.VMEM((bpw,), jnp.int32),
                       pltpu.VMEM((bpw, d), table.dtype),
                       pltpu.SemaphoreType.DMA],
    )
    def k(table_hbm, idx_hbm, out_hbm, idx_v, rows_v, sem):
        wid = lax.axis_index("s") * 2 + lax.axis_index("c")
        base = wid * bpw
        pltpu.sync_copy(idx_hbm.at[pl.ds(base, bpw)], idx_v)
        pltpu.async_copy(table_hbm.at[idx_v], rows_v, sem).wait()
        pltpu.sync_copy(rows_v, out_hbm.at[pl.ds(base, bpw)])

    return k(table, idx)


def _moe_body(be_ref, xs_ref, ew1_ref, eb1_ref, ew3_ref, eb3_ref,
              ew2_ref, eb2_ref, out_ref):
    c = pl.program_id(0)
    b = pl.program_id(1)
    rows = pl.ds(b * T, T)
    xb = xs_ref[rows, :].astype(BF16)
    w1 = ew1_ref[0].astype(BF16)
    w3 = ew3_ref[0].astype(BF16)
    h1 = lax.dot_general(xb, w1, (((1,), (1,)), ((), ())),
                         preferred_element_type=F32) + eb1_ref[0]
    h3 = lax.dot_general(xb, w3, (((1,), (1,)), ((), ())),
                         preferred_element_type=F32) + eb3_ref[0]
    h = (_silu(h1) * h3).astype(BF16)
    w2 = ew2_ref[0].astype(BF16)
    o = lax.dot_general(h, w2, (((1,), (1,)), ((), ())),
                        preferred_element_type=F32)

    @pl.when(c == 0)
    def _():
        out_ref[rows, :] = o + eb2_ref[0]

    @pl.when(c != 0)
    def _():
        out_ref[rows, :] = out_ref[rows, :] + o


def _moe(be, xs, ew1, eb1, ew3, eb3, ew2, eb2):
    grid_spec = pltpu.PrefetchScalarGridSpec(
        num_scalar_prefetch=1,
        grid=(NCH, NBLK),
        in_specs=[
            pl.BlockSpec((NPAD, D), lambda c, b, be: (0, 0)),
            pl.BlockSpec((1, CH, D), lambda c, b, be: (be[b], c, 0)),
            pl.BlockSpec((1, 1, CH), lambda c, b, be: (be[b], 0, c)),
            pl.BlockSpec((1, CH, D), lambda c, b, be: (be[b], c, 0)),
            pl.BlockSpec((1, 1, CH), lambda c, b, be: (be[b], 0, c)),
            pl.BlockSpec((1, D, CH), lambda c, b, be: (be[b], 0, c)),
            pl.BlockSpec((1, 1, D), lambda c, b, be: (be[b], 0, 0)),
        ],
        out_specs=pl.BlockSpec((NPAD, D), lambda c, b, be: (0, 0)),
    )
    return _PC(
        _moe_body,
        grid_spec=grid_spec,
        out_shape=jax.ShapeDtypeStruct((NPAD, D), F32),
    )(be, xs, ew1, eb1, ew3, eb3, ew2, eb2)


def _combine_body(y_ref, g_ref, p_ref, o_ref):
    o_ref[...] = y_ref[...] + g_ref[...] * p_ref[...]


def _combine(y_att, g, probs):
    tb = 256
    row = pl.BlockSpec((tb, D), lambda i: (i, 0))
    prow = pl.BlockSpec((tb, 1), lambda i: (i, 0))
    return _PC(
        _combine_body,
        grid=(S // tb,),
        in_specs=[row, row, prow],
        out_specs=row,
        out_shape=jax.ShapeDtypeStruct((S, D), F32),
    )(y_att, g, probs)


def kernel(x, qp_w1, qp_b1, qp_w2, qp_b2, kp_w1, kp_b1, kp_w2, kp_b2,
           vp_w1, vp_b1, vp_w2, vp_b2, out_w, out_b, ln1_g, ln1_b,
           ln2_g, ln2_b, gate_w, gate_b, ew1, eb1, ew2, eb2, ew3, eb3):
    x2d = x.reshape(S, D)
    r = lambda a: a.reshape(1, -1)
    pq, pk, v = _qkv(x2d, r(ln1_g), r(ln1_b),
                     qp_w1, r(qp_b1), qp_w2, r(qp_b2),
                     kp_w1, r(kp_b1), kp_w2, r(kp_b2),
                     vp_w1, r(vp_b1), vp_w2, r(vp_b2))
    attn = _attn(pq, pk, v)
    y_att, x2b, slot, probs, tos, be = _route(
        x2d, attn, out_w, r(out_b), r(ln2_g), r(ln2_b), gate_w, r(gate_b))
    xs = _sc_row_gather(x2b, tos.reshape(NPAD))
    ffn = _moe(be.reshape(NBLK), xs, ew1, eb1.reshape(E, 1, DFF),
               ew3, eb3.reshape(E, 1, DFF), ew2, eb2.reshape(E, 1, D))
    g = _sc_row_gather(ffn, slot.reshape(S))
    y = _combine(y_att, g, probs)
    return y.reshape(1, S, D)


# speedup vs baseline: 1.5621x; 1.1634x over previous
import functools

import jax
import jax.numpy as jnp
from jax import lax
from jax.experimental import pallas as pl
from jax.experimental.pallas import tpu as pltpu
from jax.experimental.pallas import tpu_sc as plsc

S, D, H, DK, DFF, E = 2048, 768, 12, 64, 3072, 8
T = 128
NPAD = S + E * T
NBLK = NPAD // T
CH = 512
NCH = DFF // CH
F32 = jnp.float32
BF16 = jnp.bfloat16

_PC = pl.pallas_call


def _fiota(shape, dim):
    return lax.broadcasted_iota(jnp.int32, shape, dim).astype(F32)

def _ln(x, g, b):
    m = jnp.mean(x, axis=-1, keepdims=True)
    v = jnp.mean((x - m) ** 2, axis=-1, keepdims=True)
    return (x - m) * lax.rsqrt(v + 1e-5) * g + b


def _dot_nt(a, b):
    return lax.dot_general(a, b, (((1,), (1,)), ((), ())),
                           preferred_element_type=F32)


def _dot_nn(a, b):
    return lax.dot_general(a, b, (((1,), (0,)), ((), ())),
                           preferred_element_type=F32)


def _silu(x):
    return x * jax.nn.sigmoid(x)


def _attn_body(x_ref, ln1g_ref, ln1b_ref,
               qw1_ref, qb1_ref, qw2_ref, qb2_ref,
               kw1_ref, kb1_ref, kw2_ref, kb2_ref,
               vw1_ref, vb1_ref, vw2_ref, vb2_ref,
               o_ref, pq_s, pk_s, v_s):
    tb = 256
    for ci in range(S // tb):
        sl = slice(ci * tb, (ci + 1) * tb)
        x2 = _ln(x_ref[sl, :], ln1g_ref[...], ln1b_ref[...])
        q = _silu(_dot_nt(x2, qw1_ref[...]) + qb1_ref[...]) * (
            _dot_nt(x2, qw2_ref[...]) + qb2_ref[...])
        k = _silu(_dot_nt(x2, kw1_ref[...]) + kb1_ref[...]) * (
            _dot_nt(x2, kw2_ref[...]) + kb2_ref[...])
        v = _silu(_dot_nt(x2, vw1_ref[...]) + vb1_ref[...]) * (
            _dot_nt(x2, vw2_ref[...]) + vb2_ref[...])
        pq_s[sl, :] = jnp.where(q > 0, q + 1.0, jnp.exp(q))
        pk_s[sl, :] = jnp.where(k > 0, k + 1.0, jnp.exp(k))
        v_s[sl, :] = v
    for h in range(H):
        sl = slice(h * DK, (h + 1) * DK)
        pq = pq_s[:, sl]
        pk = pk_s[:, sl]
        v = v_s[:, sl]
        kv = lax.dot_general(pk, v, (((0,), (0,)), ((), ())),
                             preferred_element_type=F32)
        ksum = jnp.sum(pk, axis=0, keepdims=True)
        qks = jnp.sum(pq * ksum, axis=1, keepdims=True)
        o_ref[:, sl] = _dot_nn(pq, kv) / (qks + 1e-6)


def _attn(x, ln1g, ln1b, qw1, qb1, qw2, qb2, kw1, kb1, kw2, kb2,
          vw1, vb1, vw2, vb2):
    full = lambda shape: pl.BlockSpec(shape, lambda: tuple(0 for _ in shape))
    return _PC(
        _attn_body,
        in_specs=[full((S, D)), full((1, D)), full((1, D)),
                  full((D, D)), full((1, D)), full((D, D)), full((1, D)),
                  full((D, D)), full((1, D)), full((D, D)), full((1, D)),
                  full((D, D)), full((1, D)), full((D, D)), full((1, D))],
        out_specs=full((S, D)),
        out_shape=jax.ShapeDtypeStruct((S, D), F32),
        scratch_shapes=[pltpu.VMEM((S, D), F32)] * 3,
    )(x, ln1g, ln1b, qw1, qb1, qw2, qb2, kw1, kb1, kw2, kb2,
      vw1, vb1, vw2, vb2)


def _route_body(x_ref, attn_ref, ow_ref, ob_ref, ln2g_ref, ln2b_ref,
                gw_ref, gb_ref,
                y_ref, x2b_ref, slot_ref, prob_ref, tos_ref, be_ref,
                ranks_ref):
    y = x_ref[...] + _dot_nt(attn_ref[...], ow_ref[...]) + ob_ref[...]
    y_ref[...] = y
    x2b = _ln(y, ln2g_ref[...], ln2b_ref[...])
    x2b_ref[...] = x2b
    logits = _dot_nt(x2b, gw_ref[...]) + gb_ref[...]
    mx = jnp.max(logits, axis=1, keepdims=True)
    iota_e = _fiota((S, E), 1)
    e_idx = jnp.min(jnp.where(logits >= mx, iota_e, float(E)),
                    axis=1, keepdims=True)
    ssum = jnp.sum(jnp.exp(logits - mx), axis=1, keepdims=True)
    topv = 1.0 / ssum
    prob_ref[...] = topv / (topv + 1e-6)
    onehot = (iota_e == e_idx).astype(F32)

    rchunk = 256
    def rank_step(ci, _):
        r0 = ci * rchunk
        row_i = r0 + _fiota((rchunk, S), 0)
        col_j = _fiota((rchunk, S), 1)
        tri = (col_j < row_i).astype(F32)
        ranks_ref[pl.ds(r0, rchunk), :] = _dot_nn(tri, onehot)
        return 0
    lax.fori_loop(0, S // rchunk, rank_step, 0)
    ranks8 = ranks_ref[...]
    rank = jnp.sum(ranks8 * onehot, axis=1, keepdims=True)

    counts = jnp.sum(onehot, axis=0, keepdims=True)
    pc = jnp.floor((counts + (T - 1.0)) * (1.0 / T)) * T
    lt = (_fiota((E, E), 0) <
          _fiota((E, E), 1)).astype(F32)
    off = _dot_nn(pc, lt)
    off_tok = jnp.sum(onehot * off, axis=1, keepdims=True)
    slot = off_tok + rank
    slot_ref[...] = slot.astype(jnp.int32)

    def tos_step(b, _):
        sl_ids = b * T + _fiota((1, T), 1)
        m = slot == sl_ids
        tok_i = _fiota((S, T), 0)
        tos = jnp.sum(jnp.where(m, tok_i, 0.0), axis=0, keepdims=True)
        has = jnp.sum(m.astype(F32), axis=0, keepdims=True) > 0
        fallback = jnp.where(sl_ids >= float(S), sl_ids - float(S), sl_ids)
        tos = jnp.where(has, tos, fallback)
        tos_ref[pl.ds(b, 1), :] = tos.astype(jnp.int32)
        return 0
    lax.fori_loop(0, NBLK, tos_step, 0)

    bstart = _fiota((NBLK, E), 0) * T
    cond = (bstart >= off) & (bstart < off + pc)
    be = jnp.sum(jnp.where(cond, _fiota((NBLK, E), 1), 0.0),
                 axis=1, keepdims=True)
    be_ref[...] = be.astype(jnp.int32)


def _route(x, attn, ow, ob, ln2g, ln2b, gw, gb):
    full = lambda shape: pl.BlockSpec(shape, lambda: tuple(0 for _ in shape))
    return _PC(
        _route_body,
        in_specs=[full((S, D)), full((S, D)), full((D, D)), full((1, D)),
                  full((1, D)), full((1, D)), full((E, D)), full((1, E))],
        out_specs=[full((S, D)), full((S, D)), full((S, 1)), full((S, 1)),
                   full((NBLK, T)), full((NBLK, 1))],
        out_shape=[jax.ShapeDtypeStruct((S, D), F32),
                   jax.ShapeDtypeStruct((S, D), F32),
                   jax.ShapeDtypeStruct((S, 1), jnp.int32),
                   jax.ShapeDtypeStruct((S, 1), F32),
                   jax.ShapeDtypeStruct((NBLK, T), jnp.int32),
                   jax.ShapeDtypeStruct((NBLK, 1), jnp.int32)],
        scratch_shapes=[pltpu.VMEM((S, E), F32)],
    )(x, attn, ow, ob, ln2g, ln2b, gw, gb)


def _sc_row_gather(table, idx):
    _, d = table.shape
    b = idx.shape[0]
    nw = 32
    bpw = b // nw
    mesh = plsc.VectorSubcoreMesh(core_axis_name="c", subcore_axis_name="s")

    @functools.partial(
        pl.kernel,
        out_type=jax.ShapeDtypeStruct((b, d), table.dtype),
        mesh=mesh,
        scratch_types=[pltpu.VMEM((bpw,), jnp.int32),
                       pltpu.VMEM((bpw, d), table.dtype),
                       pltpu.SemaphoreType.DMA],
    )
    def k(table_hbm, idx_hbm, out_hbm, idx_v, rows_v, sem):
        wid = lax.axis_index("s") * 2 + lax.axis_index("c")
        base = wid * bpw
        pltpu.sync_copy(idx_hbm.at[pl.ds(base, bpw)], idx_v)
        pltpu.async_copy(table_hbm.at[idx_v], rows_v, sem).wait()
        pltpu.sync_copy(rows_v, out_hbm.at[pl.ds(base, bpw)])

    return k(table, idx)


def _moe_body(be_ref, xs_ref, ew1_ref, eb1_ref, ew3_ref, eb3_ref,
              ew2_ref, eb2_ref, out_ref):
    c = pl.program_id(0)
    b = pl.program_id(1)
    rows = pl.ds(b * T, T)
    xb = xs_ref[rows, :].astype(BF16)
    w1 = ew1_ref[0].astype(BF16)
    w3 = ew3_ref[0].astype(BF16)
    h1 = lax.dot_general(xb, w1, (((1,), (1,)), ((), ())),
                         preferred_element_type=F32) + eb1_ref[0]
    h3 = lax.dot_general(xb, w3, (((1,), (1,)), ((), ())),
                         preferred_element_type=F32) + eb3_ref[0]
    h = (_silu(h1) * h3).astype(BF16)
    w2 = ew2_ref[0].astype(BF16)
    o = lax.dot_general(h, w2, (((1,), (1,)), ((), ())),
                        preferred_element_type=F32)

    @pl.when(c == 0)
    def _():
        out_ref[rows, :] = o + eb2_ref[0]

    @pl.when(c != 0)
    def _():
        out_ref[rows, :] = out_ref[rows, :] + o


def _moe(be, xs, ew1, eb1, ew3, eb3, ew2, eb2):
    grid_spec = pltpu.PrefetchScalarGridSpec(
        num_scalar_prefetch=1,
        grid=(NCH, NBLK),
        in_specs=[
            pl.BlockSpec((NPAD, D), lambda c, b, be: (0, 0)),
            pl.BlockSpec((1, CH, D), lambda c, b, be: (be[b], c, 0)),
            pl.BlockSpec((1, 1, CH), lambda c, b, be: (be[b], 0, c)),
            pl.BlockSpec((1, CH, D), lambda c, b, be: (be[b], c, 0)),
            pl.BlockSpec((1, 1, CH), lambda c, b, be: (be[b], 0, c)),
            pl.BlockSpec((1, D, CH), lambda c, b, be: (be[b], 0, c)),
            pl.BlockSpec((1, 1, D), lambda c, b, be: (be[b], 0, 0)),
        ],
        out_specs=pl.BlockSpec((NPAD, D), lambda c, b, be: (0, 0)),
    )
    return _PC(
        _moe_body,
        grid_spec=grid_spec,
        out_shape=jax.ShapeDtypeStruct((NPAD, D), F32),
    )(be, xs, ew1, eb1, ew3, eb3, ew2, eb2)


def _combine_body(y_ref, g_ref, p_ref, o_ref):
    o_ref[...] = y_ref[...] + g_ref[...] * p_ref[...]


def _combine(y_att, g, probs):
    tb = 256
    row = pl.BlockSpec((tb, D), lambda i: (i, 0))
    prow = pl.BlockSpec((tb, 1), lambda i: (i, 0))
    return _PC(
        _combine_body,
        grid=(S // tb,),
        in_specs=[row, row, prow],
        out_specs=row,
        out_shape=jax.ShapeDtypeStruct((S, D), F32),
    )(y_att, g, probs)


def kernel(x, qp_w1, qp_b1, qp_w2, qp_b2, kp_w1, kp_b1, kp_w2, kp_b2,
           vp_w1, vp_b1, vp_w2, vp_b2, out_w, out_b, ln1_g, ln1_b,
           ln2_g, ln2_b, gate_w, gate_b, ew1, eb1, ew2, eb2, ew3, eb3):
    x2d = x.reshape(S, D)
    r = lambda a: a.reshape(1, -1)
    attn = _attn(x2d, r(ln1_g), r(ln1_b),
                 qp_w1, r(qp_b1), qp_w2, r(qp_b2),
                 kp_w1, r(kp_b1), kp_w2, r(kp_b2),
                 vp_w1, r(vp_b1), vp_w2, r(vp_b2))
    y_att, x2b, slot, probs, tos, be = _route(
        x2d, attn, out_w, r(out_b), r(ln2_g), r(ln2_b), gate_w, r(gate_b))
    xs = _sc_row_gather(x2b, tos.reshape(NPAD))
    ffn = _moe(be.reshape(NBLK), xs, ew1, eb1.reshape(E, 1, DFF),
               ew3, eb3.reshape(E, 1, DFF), ew2, eb2.reshape(E, 1, D))
    g = _sc_row_gather(ffn, slot.reshape(S))
    y = _combine(y_att, g, probs)
    return y.reshape(1, S, D)
